# baseline (speedup 1.0000x reference)
import functools

import jax
import jax.numpy as jnp
from jax import lax
from jax.experimental import pallas as pl
from jax.experimental.pallas import tpu as pltpu
from jax.experimental.pallas import tpu_sc as plsc

N = 10000
E = 320000
D = 128
EPS = 1e-5

NC = 2
NS = 16
NW = NC * NS
EPW = E // NW
CHUNK = 104
NCF = EPW // CHUNK
TAIL = EPW - NCF * CHUNK
ROWS_A = 624
ZROWS = 104
REM = N - NS * ROWS_A
DEGW = 16


@functools.cache
def _make_deg_kernel():
    return functools.partial(
        pl.kernel,
        mesh=plsc.VectorSubcoreMesh(core_axis_name="c", subcore_axis_name="s"),
        compiler_params=pltpu.CompilerParams(use_tc_tiling_on_sc=False),
        out_type=jax.ShapeDtypeStruct((NC, N, DEGW), jnp.float32),
        scratch_types=[
            pltpu.VMEM((EPW,), jnp.int32),
            pltpu.VMEM((CHUNK, DEGW), jnp.float32),
            pltpu.VMEM((ROWS_A, DEGW), jnp.float32),
            pltpu.VMEM_SHARED((N, DEGW), jnp.float32),
        ],
    )(_deg_body)


def _deg_body(adj_hbm, out_hbm, dstidx_v, ones_v, zero_v, deg_sh):
    c = lax.axis_index("c")
    s = lax.axis_index("s")
    w = c * NS + s

    def fill(i, carry):
        ones_v[i, pl.ds(0, 16)] = jnp.ones((16,), jnp.float32)
        return carry

    lax.fori_loop(0, CHUNK, fill, 0)

    def zrow(i, carry):
        zero_v[i, pl.ds(0, 16)] = jnp.zeros((16,), jnp.float32)
        return carry

    lax.fori_loop(0, ROWS_A, zrow, 0)
    pltpu.sync_copy(zero_v, deg_sh.at[pl.ds(s * ROWS_A, ROWS_A)])

    @pl.when(s == 0)
    def _():
        pltpu.sync_copy(zero_v.at[pl.ds(0, REM)],
                        deg_sh.at[pl.ds(NS * ROWS_A, REM)])

    pltpu.sync_copy(adj_hbm.at[1].at[pl.ds(w * EPW, EPW)], dstidx_v)
    plsc.subcore_barrier()

    def body(j, carry):
        pltpu.sync_copy(ones_v,
                        deg_sh.at[dstidx_v.at[pl.ds(j * CHUNK, CHUNK)]],
                        add=True)
        return carry

    lax.fori_loop(0, NCF, body, 0)
    pltpu.sync_copy(ones_v.at[pl.ds(0, TAIL)],
                    deg_sh.at[dstidx_v.at[pl.ds(NCF * CHUNK, TAIL)]],
                    add=True)
    plsc.subcore_barrier()
    pltpu.sync_copy(deg_sh.at[pl.ds(s * ROWS_A, ROWS_A)],
                    out_hbm.at[c].at[pl.ds(s * ROWS_A, ROWS_A)])

    @pl.when(s == 0)
    def _():
        pltpu.sync_copy(deg_sh.at[pl.ds(NS * ROWS_A, REM)],
                        out_hbm.at[c].at[pl.ds(NS * ROWS_A, REM)])


@functools.cache
def _make_edge_kernel():
    return functools.partial(
        pl.kernel,
        mesh=plsc.VectorSubcoreMesh(core_axis_name="c", subcore_axis_name="s"),
        compiler_params=pltpu.CompilerParams(use_tc_tiling_on_sc=False),
        out_type=jax.ShapeDtypeStruct((NC, N, D), jnp.float32),
        scratch_types=[
            pltpu.VMEM((EPW,), jnp.int32),
            pltpu.VMEM((EPW,), jnp.int32),
            pltpu.VMEM((CHUNK, D), jnp.float32),
            pltpu.VMEM((CHUNK, D), jnp.float32),
            pltpu.VMEM_SHARED((N, D), jnp.float32),
            pltpu.SemaphoreType.DMA,
            pltpu.SemaphoreType.DMA,
        ],
    )(_edge_body)


NB = 2


def _edge_body(h2_hbm, adj_hbm, out_hbm,
               srcidx_v, dstidx_v, r0, r1, acc_sh, g0, g1):
    bufs = (r0, r1)
    gsems = (g0, g1)
    c = lax.axis_index("c")
    s = lax.axis_index("s")
    w = c * NS + s

    pltpu.async_copy(adj_hbm.at[0].at[pl.ds(w * EPW, EPW)], srcidx_v, gsems[0])
    pltpu.async_copy(adj_hbm.at[1].at[pl.ds(w * EPW, EPW)], dstidx_v, gsems[1])

    def zrow(i, carry):
        for j in range(D // 16):
            r1[i, pl.ds(j * 16, 16)] = jnp.zeros((16,), jnp.float32)
        return carry

    lax.fori_loop(0, 96, zrow, 0)

    def zacc(k, carry):
        pltpu.sync_copy(r1.at[pl.ds(0, 96)],
                        acc_sh.at[pl.ds(s * ROWS_A + k * 96, 96)])
        return carry

    lax.fori_loop(0, 6, zacc, 0)
    pltpu.sync_copy(r1.at[pl.ds(0, 48)],
                    acc_sh.at[pl.ds(s * ROWS_A + 576, 48)])

    @pl.when(s == 0)
    def _():
        pltpu.sync_copy(r1.at[pl.ds(0, REM)],
                        acc_sh.at[pl.ds(NS * ROWS_A, REM)])

    pltpu.make_async_copy(adj_hbm.at[0].at[pl.ds(w * EPW, EPW)], srcidx_v,
                          gsems[0]).wait()
    pltpu.make_async_copy(adj_hbm.at[1].at[pl.ds(w * EPW, EPW)], dstidx_v,
                          gsems[1]).wait()
    plsc.subcore_barrier()

    def sidx(j):
        return srcidx_v.at[pl.ds(j * CHUNK, CHUNK)]

    def didx(j):
        return dstidx_v.at[pl.ds(j * CHUNK, CHUNK)]

    for b in range(NB):
        pltpu.async_copy(h2_hbm.at[sidx(b)], bufs[b], gsems[b])

    def body(g, carry):
        for b in range(NB):
            j = g * NB + b
            pltpu.make_async_copy(h2_hbm.at[sidx(j)], bufs[b],
                                  gsems[b]).wait()
            pltpu.sync_copy(bufs[b], acc_sh.at[didx(j)], add=True)

            @pl.when(j + NB < NCF)
            def _():
                pltpu.async_copy(h2_hbm.at[sidx(j + NB)], bufs[b], gsems[b])
        return carry

    lax.fori_loop(0, NCF // NB, body, 0)
    pltpu.async_copy(h2_hbm.at[srcidx_v.at[pl.ds(NCF * CHUNK, TAIL)]],
                     bufs[0].at[pl.ds(0, TAIL)], gsems[0])
    pltpu.make_async_copy(h2_hbm.at[srcidx_v.at[pl.ds(NCF * CHUNK, TAIL)]],
                          bufs[0].at[pl.ds(0, TAIL)], gsems[0]).wait()
    pltpu.sync_copy(bufs[0].at[pl.ds(0, TAIL)],
                    acc_sh.at[dstidx_v.at[pl.ds(NCF * CHUNK, TAIL)]],
                    add=True)
    plsc.subcore_barrier()
    pltpu.sync_copy(acc_sh.at[pl.ds(s * ROWS_A, ROWS_A)],
                    out_hbm.at[c].at[pl.ds(s * ROWS_A, ROWS_A)])

    @pl.when(s == 0)
    def _():
        pltpu.sync_copy(acc_sh.at[pl.ds(NS * ROWS_A, REM)],
                        out_hbm.at[c].at[pl.ds(NS * ROWS_A, REM)])


def _t0_body(x_ref, w_ref, degp_ref, h2_ref):
    deg = degp_ref[0][:, :1] + degp_ref[1][:, :1] + 1.0
    dinv = lax.rsqrt(jnp.maximum(deg, 1.0))
    h = jnp.dot(x_ref[...], w_ref[...], preferred_element_type=jnp.float32)
    h2_ref[...] = h * dinv


NBLK = 10
RB = N // NBLK


def _bn_phase0(accp_ref, h2p_ref, degp_ref, b_ref, rb, t_scr, sum_scr, sq_scr):
    deg = degp_ref[0][:, :1] + degp_ref[1][:, :1] + 1.0
    dinv = lax.rsqrt(jnp.maximum(deg, 1.0))
    t = (accp_ref[0] + accp_ref[1] + h2p_ref[...]) * dinv + b_ref[...]
    t_scr[pl.ds(rb * RB, RB), :] = t
    ps = jnp.sum(t, axis=0, keepdims=True)
    pq = jnp.sum(t * t, axis=0, keepdims=True)

    @pl.when(rb == 0)
    def _():
        sum_scr[...] = ps
        sq_scr[...] = pq

    @pl.when(rb > 0)
    def _():
        sum_scr[...] = sum_scr[...] + ps
        sq_scr[...] = sq_scr[...] + pq


def _bn_phase1_hact(degp_ref, g_ref, be_ref, rb, t_scr, sum_scr, sq_scr):
    mean = sum_scr[...] * (1.0 / N)
    var = sq_scr[...] * (1.0 / N) - mean * mean
    t = t_scr[pl.ds(rb * RB, RB), :]
    hact = jnp.maximum((t - mean) * lax.rsqrt(var + EPS) * g_ref[...]
                       + be_ref[...], 0.0)
    deg = degp_ref[0][:, :1] + degp_ref[1][:, :1] + 1.0
    dinv = lax.rsqrt(jnp.maximum(deg, 1.0))
    return hact, dinv


def _mid_body(accp_ref, h2p_ref, degp_ref, b_ref, g_ref, be_ref, w_ref,
              hact_ref, h2_ref, t_scr, sum_scr, sq_scr):
    ph = pl.program_id(0)
    rb = pl.program_id(1)

    @pl.when(ph == 0)
    def _():
        _bn_phase0(accp_ref, h2p_ref, degp_ref, b_ref, rb,
                   t_scr, sum_scr, sq_scr)

    @pl.when(ph == 1)
    def _():
        hact, dinv = _bn_phase1_hact(degp_ref, g_ref, be_ref, rb,
                                     t_scr, sum_scr, sq_scr)
        hact_ref[...] = hact
        h2_ref[...] = jnp.dot(hact, w_ref[...],
                              preferred_element_type=jnp.float32) * dinv


def _fin_body(accp_ref, h2p_ref, degp_ref, b_ref, g_ref, be_ref,
              hact1_ref, hact2_ref, wo_ref, bo_ref, out_ref,
              t_scr, sum_scr, sq_scr):
    ph = pl.program_id(0)
    rb = pl.program_id(1)

    @pl.when(ph == 0)
    def _():
        _bn_phase0(accp_ref, h2p_ref, degp_ref, b_ref, rb,
                   t_scr, sum_scr, sq_scr)

    @pl.when(ph == 1)
    def _():
        hact3, _ = _bn_phase1_hact(degp_ref, g_ref, be_ref, rb,
                                   t_scr, sum_scr, sq_scr)
        hj = jnp.maximum(jnp.maximum(hact1_ref[...], hact2_ref[...]), hact3)
        o = jnp.dot(hj, wo_ref[...],
                    preferred_element_type=jnp.float32) + bo_ref[...]
        m = jnp.max(o, axis=1, keepdims=True)
        sh = o - m
        lse = jnp.log(jnp.sum(jnp.exp(sh), axis=1, keepdims=True))
        out_ref[...] = sh - lse


def _row_block(pin_phase1):
    if pin_phase1:
        return lambda ph, rb: (rb * (1 - ph), 0)
    return lambda ph, rb: (rb, 0)


def _row_block3(pin_phase1):
    if pin_phase1:
        return lambda ph, rb: (0, rb * (1 - ph), 0)
    return lambda ph, rb: (0, rb, 0)


_BN_SCRATCH = [
    pltpu.VMEM((N, D), jnp.float32),
    pltpu.VMEM((1, D), jnp.float32),
    pltpu.VMEM((1, D), jnp.float32),
]

_ARB2 = pltpu.CompilerParams(
    dimension_semantics=("arbitrary", "arbitrary"))

_mid_in_specs = [
    pl.BlockSpec((2, RB, D), _row_block3(True)),
    pl.BlockSpec((RB, D), _row_block(True)),
    pl.BlockSpec((2, RB, DEGW), _row_block3(False)),
    pl.BlockSpec((D,), lambda ph, rb: (0,)),
    pl.BlockSpec((D,), lambda ph, rb: (0,)),
    pl.BlockSpec((D,), lambda ph, rb: (0,)),
    pl.BlockSpec((D, D), lambda ph, rb: (0, 0)),
]

_mid_call = pl.pallas_call(
    _mid_body,
    grid=(2, NBLK),
    in_specs=_mid_in_specs,
    out_specs=[pl.BlockSpec((RB, D), lambda ph, rb: (rb * ph, 0)),
               pl.BlockSpec((RB, D), lambda ph, rb: (rb * ph, 0))],
    out_shape=[jax.ShapeDtypeStruct((N, D), jnp.float32),
               jax.ShapeDtypeStruct((N, D), jnp.float32)],
    scratch_shapes=_BN_SCRATCH,
    compiler_params=_ARB2,
)

_fin_in_specs = [
    pl.BlockSpec((2, RB, D), _row_block3(True)),
    pl.BlockSpec((RB, D), _row_block(True)),
    pl.BlockSpec((2, RB, DEGW), _row_block3(False)),
    pl.BlockSpec((D,), lambda ph, rb: (0,)),
    pl.BlockSpec((D,), lambda ph, rb: (0,)),
    pl.BlockSpec((D,), lambda ph, rb: (0,)),
    pl.BlockSpec((RB, D), lambda ph, rb: (rb * ph, 0)),
    pl.BlockSpec((RB, D), lambda ph, rb: (rb * ph, 0)),
    pl.BlockSpec((D, D), lambda ph, rb: (0, 0)),
    pl.BlockSpec((D,), lambda ph, rb: (0,)),
]

_fin_call = pl.pallas_call(
    _fin_body,
    grid=(2, NBLK),
    in_specs=_fin_in_specs,
    out_specs=pl.BlockSpec((RB, D), lambda ph, rb: (rb * ph, 0)),
    out_shape=jax.ShapeDtypeStruct((N, D), jnp.float32),
    scratch_shapes=_BN_SCRATCH,
    compiler_params=_ARB2,
)

_t0_call = pl.pallas_call(
    _t0_body,
    grid=(NBLK,),
    in_specs=[pl.BlockSpec((RB, D), lambda rb: (rb, 0)),
              pl.BlockSpec((D, D), lambda rb: (0, 0)),
              pl.BlockSpec((2, RB, DEGW), lambda rb: (0, rb, 0))],
    out_specs=pl.BlockSpec((RB, D), lambda rb: (rb, 0)),
    out_shape=jax.ShapeDtypeStruct((N, D), jnp.float32),
)


def kernel(x, adj_m, W0, b0, g0, be0, W1, b1, g1, be1, W2, b2, g2, be2, Wo, bo):
    degp = _make_deg_kernel()(adj_m)
    h2 = _t0_call(x, W0, degp)

    hacts = []
    for (b, g, be, Wn) in ((b0, g0, be0, W1), (b1, g1, be1, W2)):
        accp = _make_edge_kernel()(h2, adj_m)
        hact, h2 = _mid_call(accp, h2, degp, b, g, be, Wn)
        hacts.append(hact)

    accp = _make_edge_kernel()(h2, adj_m)
    return _fin_call(accp, h2, degp, b2, g2, be2, hacts[0], hacts[1], Wo, bo)


# speedup vs baseline: 1.0525x; 1.0525x over previous
import functools

import jax
import jax.numpy as jnp
from jax import lax
from jax.experimental import pallas as pl
from jax.experimental.pallas import tpu as pltpu
from jax.experimental.pallas import tpu_sc as plsc

N = 10000
E = 320000
D = 128
EPS = 1e-5

NC = 2
NS = 16
NW = NC * NS
EPW = E // NW
CHUNK = 104
NCF = EPW // CHUNK
TAIL = EPW - NCF * CHUNK
ROWS_A = 624
ZROWS = 104
REM = N - NS * ROWS_A
DEGW = 16


@functools.cache
def _make_deg_kernel():
    return functools.partial(
        pl.kernel,
        mesh=plsc.VectorSubcoreMesh(core_axis_name="c", subcore_axis_name="s"),
        compiler_params=pltpu.CompilerParams(use_tc_tiling_on_sc=False),
        out_type=jax.ShapeDtypeStruct((NC, N, DEGW), jnp.float32),
        scratch_types=[
            pltpu.VMEM((EPW,), jnp.int32),
            pltpu.VMEM((CHUNK, DEGW), jnp.float32),
            pltpu.VMEM((ROWS_A, DEGW), jnp.float32),
            pltpu.VMEM_SHARED((N, DEGW), jnp.float32),
        ],
    )(_deg_body)


def _deg_body(adj_hbm, out_hbm, dstidx_v, ones_v, zero_v, deg_sh):
    c = lax.axis_index("c")
    s = lax.axis_index("s")
    w = c * NS + s

    def fill(i, carry):
        ones_v[i, pl.ds(0, 16)] = jnp.ones((16,), jnp.float32)
        return carry

    lax.fori_loop(0, CHUNK, fill, 0)

    def zrow(i, carry):
        zero_v[i, pl.ds(0, 16)] = jnp.zeros((16,), jnp.float32)
        return carry

    lax.fori_loop(0, ROWS_A, zrow, 0)
    pltpu.sync_copy(zero_v, deg_sh.at[pl.ds(s * ROWS_A, ROWS_A)])

    @pl.when(s == 0)
    def _():
        pltpu.sync_copy(zero_v.at[pl.ds(0, REM)],
                        deg_sh.at[pl.ds(NS * ROWS_A, REM)])

    pltpu.sync_copy(adj_hbm.at[1].at[pl.ds(w * EPW, EPW)], dstidx_v)
    plsc.subcore_barrier()

    def body(j, carry):
        pltpu.sync_copy(ones_v,
                        deg_sh.at[dstidx_v.at[pl.ds(j * CHUNK, CHUNK)]],
                        add=True)
        return carry

    lax.fori_loop(0, NCF, body, 0)
    pltpu.sync_copy(ones_v.at[pl.ds(0, TAIL)],
                    deg_sh.at[dstidx_v.at[pl.ds(NCF * CHUNK, TAIL)]],
                    add=True)
    plsc.subcore_barrier()
    pltpu.sync_copy(deg_sh.at[pl.ds(s * ROWS_A, ROWS_A)],
                    out_hbm.at[c].at[pl.ds(s * ROWS_A, ROWS_A)])

    @pl.when(s == 0)
    def _():
        pltpu.sync_copy(deg_sh.at[pl.ds(NS * ROWS_A, REM)],
                        out_hbm.at[c].at[pl.ds(NS * ROWS_A, REM)])


@functools.cache
def _make_edge_kernel():
    return functools.partial(
        pl.kernel,
        mesh=plsc.VectorSubcoreMesh(core_axis_name="c", subcore_axis_name="s"),
        compiler_params=pltpu.CompilerParams(use_tc_tiling_on_sc=False),
        out_type=jax.ShapeDtypeStruct((NC, N, D), jnp.float32),
        scratch_types=[
            pltpu.VMEM((EPW,), jnp.int32),
            pltpu.VMEM((EPW,), jnp.int32),
            pltpu.VMEM((CHUNK, D), jnp.float32),
            pltpu.VMEM((CHUNK, D), jnp.float32),
            pltpu.VMEM_SHARED((N, D), jnp.float32),
            pltpu.SemaphoreType.DMA,
            pltpu.SemaphoreType.DMA,
        ],
    )(_edge_body)


NB = 2


def _edge_body(h2_hbm, adj_hbm, out_hbm,
               srcidx_v, dstidx_v, r0, r1, acc_sh, g0, g1):
    bufs = (r0, r1)
    gsems = (g0, g1)
    c = lax.axis_index("c")
    s = lax.axis_index("s")
    w = c * NS + s

    pltpu.async_copy(adj_hbm.at[0].at[pl.ds(w * EPW, EPW)], srcidx_v, gsems[0])
    pltpu.async_copy(adj_hbm.at[1].at[pl.ds(w * EPW, EPW)], dstidx_v, gsems[1])

    def zrow(i, carry):
        for j in range(D // 16):
            r1[i, pl.ds(j * 16, 16)] = jnp.zeros((16,), jnp.float32)
        return carry

    lax.fori_loop(0, 96, zrow, 0)

    def zacc(k, carry):
        pltpu.sync_copy(r1.at[pl.ds(0, 96)],
                        acc_sh.at[pl.ds(s * ROWS_A + k * 96, 96)])
        return carry

    lax.fori_loop(0, 6, zacc, 0)
    pltpu.sync_copy(r1.at[pl.ds(0, 48)],
                    acc_sh.at[pl.ds(s * ROWS_A + 576, 48)])

    @pl.when(s == 0)
    def _():
        pltpu.sync_copy(r1.at[pl.ds(0, REM)],
                        acc_sh.at[pl.ds(NS * ROWS_A, REM)])

    pltpu.make_async_copy(adj_hbm.at[0].at[pl.ds(w * EPW, EPW)], srcidx_v,
                          gsems[0]).wait()
    pltpu.make_async_copy(adj_hbm.at[1].at[pl.ds(w * EPW, EPW)], dstidx_v,
                          gsems[1]).wait()
    plsc.subcore_barrier()

    def sidx(j):
        return srcidx_v.at[pl.ds(j * CHUNK, CHUNK)]

    def didx(j):
        return dstidx_v.at[pl.ds(j * CHUNK, CHUNK)]

    for b in range(NB):
        pltpu.async_copy(h2_hbm.at[sidx(b)], bufs[b], gsems[b])

    def body(g, carry):
        for b in range(NB):
            j = g * NB + b
            pltpu.make_async_copy(h2_hbm.at[sidx(j)], bufs[b],
                                  gsems[b]).wait()
            pltpu.sync_copy(bufs[b], acc_sh.at[didx(j)], add=True)

            @pl.when(j + NB < NCF)
            def _():
                pltpu.async_copy(h2_hbm.at[sidx(j + NB)], bufs[b], gsems[b])
        return carry

    lax.fori_loop(0, NCF // NB, body, 0)
    pltpu.async_copy(h2_hbm.at[srcidx_v.at[pl.ds(NCF * CHUNK, TAIL)]],
                     bufs[0].at[pl.ds(0, TAIL)], gsems[0])
    pltpu.make_async_copy(h2_hbm.at[srcidx_v.at[pl.ds(NCF * CHUNK, TAIL)]],
                          bufs[0].at[pl.ds(0, TAIL)], gsems[0]).wait()
    pltpu.sync_copy(bufs[0].at[pl.ds(0, TAIL)],
                    acc_sh.at[dstidx_v.at[pl.ds(NCF * CHUNK, TAIL)]],
                    add=True)
    plsc.subcore_barrier()
    pltpu.sync_copy(acc_sh.at[pl.ds(s * ROWS_A, ROWS_A)],
                    out_hbm.at[c].at[pl.ds(s * ROWS_A, ROWS_A)])

    @pl.when(s == 0)
    def _():
        pltpu.sync_copy(acc_sh.at[pl.ds(NS * ROWS_A, REM)],
                        out_hbm.at[c].at[pl.ds(NS * ROWS_A, REM)])


def _t0_body(x_ref, w_ref, degp_ref, h2_ref):
    deg = degp_ref[0][:, :1] + degp_ref[1][:, :1] + 1.0
    dinv = lax.rsqrt(jnp.maximum(deg, 1.0))
    h = jnp.dot(x_ref[...], w_ref[...], preferred_element_type=jnp.float32)
    h2_ref[...] = h * dinv


def _mid_body(accp_ref, h2p_ref, degp_ref, b_ref, g_ref, be_ref,
              w_ref, hact_ref, h2_ref):
    deg = degp_ref[0][:, :1] + degp_ref[1][:, :1] + 1.0
    dinv = lax.rsqrt(jnp.maximum(deg, 1.0))
    t = (accp_ref[0] + accp_ref[1] + h2p_ref[...]) * dinv + b_ref[...]
    mean = jnp.mean(t, axis=0, keepdims=True)
    var = jnp.mean((t - mean) ** 2, axis=0, keepdims=True)
    hact = jnp.maximum((t - mean) * lax.rsqrt(var + EPS) * g_ref[...]
                       + be_ref[...], 0.0)
    hact_ref[...] = hact
    h2_ref[...] = jnp.dot(hact, w_ref[...],
                          preferred_element_type=jnp.float32) * dinv


def _fin_body(accp_ref, h2p_ref, degp_ref, b_ref, g_ref, be_ref,
              hact1_ref, hact2_ref, wo_ref, bo_ref, out_ref):
    deg = degp_ref[0][:, :1] + degp_ref[1][:, :1] + 1.0
    dinv = lax.rsqrt(jnp.maximum(deg, 1.0))
    t = (accp_ref[0] + accp_ref[1] + h2p_ref[...]) * dinv + b_ref[...]
    mean = jnp.mean(t, axis=0, keepdims=True)
    var = jnp.mean((t - mean) ** 2, axis=0, keepdims=True)
    hact3 = jnp.maximum((t - mean) * lax.rsqrt(var + EPS) * g_ref[...]
                        + be_ref[...], 0.0)
    hj = jnp.maximum(jnp.maximum(hact1_ref[...], hact2_ref[...]), hact3)
    o = jnp.dot(hj, wo_ref[...], preferred_element_type=jnp.float32) + bo_ref[...]
    m = jnp.max(o, axis=1, keepdims=True)
    sh = o - m
    lse = jnp.log(jnp.sum(jnp.exp(sh), axis=1, keepdims=True))
    out_ref[...] = sh - lse


_t0_call = pl.pallas_call(
    _t0_body,
    out_shape=jax.ShapeDtypeStruct((N, D), jnp.float32),
)

_mid_call = pl.pallas_call(
    _mid_body,
    out_shape=[jax.ShapeDtypeStruct((N, D), jnp.float32),
               jax.ShapeDtypeStruct((N, D), jnp.float32)],
)

_fin_call = pl.pallas_call(
    _fin_body,
    out_shape=jax.ShapeDtypeStruct((N, D), jnp.float32),
)


def kernel(x, adj_m, W0, b0, g0, be0, W1, b1, g1, be1, W2, b2, g2, be2, Wo, bo):
    degp = _make_deg_kernel()(adj_m)
    h2 = _t0_call(x, W0, degp)

    hacts = []
    for (b, g, be, Wn) in ((b0, g0, be0, W1), (b1, g1, be1, W2)):
        accp = _make_edge_kernel()(h2, adj_m)
        hact, h2 = _mid_call(accp, h2, degp, b, g, be, Wn)
        hacts.append(hact)

    accp = _make_edge_kernel()(h2, adj_m)
    return _fin_call(accp, h2, degp, b2, g2, be2, hacts[0], hacts[1], Wo, bo)


# speedup vs baseline: 1.0539x; 1.0013x over previous
import functools

import jax
import jax.numpy as jnp
from jax import lax
from jax.experimental import pallas as pl
from jax.experimental.pallas import tpu as pltpu
from jax.experimental.pallas import tpu_sc as plsc

N = 10000
E = 320000
D = 128
EPS = 1e-5

NC = 2
NS = 16
NW = NC * NS
EPW = E // NW
CHUNK = 104
NCF = EPW // CHUNK
TAIL = EPW - NCF * CHUNK
ROWS_A = 624
REM = N - NS * ROWS_A
DEGW = 16


@functools.cache
def _make_deg_kernel():
    return functools.partial(
        pl.kernel,
        mesh=plsc.VectorSubcoreMesh(core_axis_name="c", subcore_axis_name="s"),
        compiler_params=pltpu.CompilerParams(use_tc_tiling_on_sc=False),
        out_type=jax.ShapeDtypeStruct((NC, N, DEGW), jnp.float32),
        scratch_types=[
            pltpu.VMEM((EPW,), jnp.int32),
            pltpu.VMEM((CHUNK, DEGW), jnp.float32),
            pltpu.VMEM((ROWS_A, DEGW), jnp.float32),
            pltpu.VMEM_SHARED((N, DEGW), jnp.float32),
        ],
    )(_deg_body)


def _deg_body(adj_hbm, out_hbm, dstidx_v, ones_v, zero_v, deg_sh):
    c = lax.axis_index("c")
    s = lax.axis_index("s")
    w = c * NS + s

    def fill(i, carry):
        ones_v[i, pl.ds(0, 16)] = jnp.ones((16,), jnp.float32)
        return carry

    lax.fori_loop(0, CHUNK, fill, 0)

    def zrow(i, carry):
        zero_v[i, pl.ds(0, 16)] = jnp.zeros((16,), jnp.float32)
        return carry

    lax.fori_loop(0, ROWS_A, zrow, 0)
    pltpu.sync_copy(zero_v, deg_sh.at[pl.ds(s * ROWS_A, ROWS_A)])

    @pl.when(s == 0)
    def _():
        pltpu.sync_copy(zero_v.at[pl.ds(0, REM)],
                        deg_sh.at[pl.ds(NS * ROWS_A, REM)])

    pltpu.sync_copy(adj_hbm.at[1].at[pl.ds(w * EPW, EPW)], dstidx_v)
    plsc.subcore_barrier()

    def body(j, carry):
        pltpu.sync_copy(ones_v,
                        deg_sh.at[dstidx_v.at[pl.ds(j * CHUNK, CHUNK)]],
                        add=True)
        return carry

    lax.fori_loop(0, NCF, body, 0)
    pltpu.sync_copy(ones_v.at[pl.ds(0, TAIL)],
                    deg_sh.at[dstidx_v.at[pl.ds(NCF * CHUNK, TAIL)]],
                    add=True)
    plsc.subcore_barrier()
    pltpu.sync_copy(deg_sh.at[pl.ds(s * ROWS_A, ROWS_A)],
                    out_hbm.at[c].at[pl.ds(s * ROWS_A, ROWS_A)])

    @pl.when(s == 0)
    def _():
        pltpu.sync_copy(deg_sh.at[pl.ds(NS * ROWS_A, REM)],
                        out_hbm.at[c].at[pl.ds(NS * ROWS_A, REM)])


@functools.cache
def _make_edge_kernel():
    return functools.partial(
        pl.kernel,
        mesh=plsc.VectorSubcoreMesh(core_axis_name="c", subcore_axis_name="s"),
        compiler_params=pltpu.CompilerParams(use_tc_tiling_on_sc=False),
        out_type=jax.ShapeDtypeStruct((NC, N, D), jnp.float32),
        scratch_types=[
            pltpu.VMEM((EPW,), jnp.int32),
            pltpu.VMEM((EPW,), jnp.int32),
            pltpu.VMEM((CHUNK, D), jnp.float32),
            pltpu.VMEM((CHUNK, D), jnp.float32),
            pltpu.VMEM_SHARED((N, D), jnp.float32),
            pltpu.SemaphoreType.DMA,
            pltpu.SemaphoreType.DMA,
        ],
    )(_edge_body)


NB = 2


def _edge_body(h2_hbm, adj_hbm, out_hbm,
               srcidx_v, dstidx_v, r0, r1, acc_sh, g0, g1):
    bufs = (r0, r1)
    gsems = (g0, g1)
    c = lax.axis_index("c")
    s = lax.axis_index("s")
    w = c * NS + s

    pltpu.async_copy(adj_hbm.at[0].at[pl.ds(w * EPW, EPW)], srcidx_v, gsems[0])
    pltpu.async_copy(adj_hbm.at[1].at[pl.ds(w * EPW, EPW)], dstidx_v, gsems[1])

    def zrow(i, carry):
        for j in range(D // 16):
            r1[i, pl.ds(j * 16, 16)] = jnp.zeros((16,), jnp.float32)
        return carry

    lax.fori_loop(0, 96, zrow, 0)

    def zacc(k, carry):
        pltpu.sync_copy(r1.at[pl.ds(0, 96)],
                        acc_sh.at[pl.ds(s * ROWS_A + k * 96, 96)])
        return carry

    lax.fori_loop(0, 6, zacc, 0)
    pltpu.sync_copy(r1.at[pl.ds(0, 48)],
                    acc_sh.at[pl.ds(s * ROWS_A + 576, 48)])

    @pl.when(s == 0)
    def _():
        pltpu.sync_copy(r1.at[pl.ds(0, REM)],
                        acc_sh.at[pl.ds(NS * ROWS_A, REM)])

    pltpu.make_async_copy(adj_hbm.at[0].at[pl.ds(w * EPW, EPW)], srcidx_v,
                          gsems[0]).wait()
    pltpu.make_async_copy(adj_hbm.at[1].at[pl.ds(w * EPW, EPW)], dstidx_v,
                          gsems[1]).wait()
    plsc.subcore_barrier()

    def sidx(j):
        return srcidx_v.at[pl.ds(j * CHUNK, CHUNK)]

    def didx(j):
        return dstidx_v.at[pl.ds(j * CHUNK, CHUNK)]

    for b in range(NB):
        pltpu.async_copy(h2_hbm.at[sidx(b)], bufs[b], gsems[b])

    def body(g, carry):
        for b in range(NB):
            j = g * NB + b
            pltpu.make_async_copy(h2_hbm.at[sidx(j)], bufs[b],
                                  gsems[b]).wait()
            pltpu.sync_copy(bufs[b], acc_sh.at[didx(j)], add=True)

            @pl.when(j + NB < NCF)
            def _():
                pltpu.async_copy(h2_hbm.at[sidx(j + NB)], bufs[b], gsems[b])
        return carry

    lax.fori_loop(0, NCF // NB, body, 0)
    pltpu.async_copy(h2_hbm.at[srcidx_v.at[pl.ds(NCF * CHUNK, TAIL)]],
                     bufs[0].at[pl.ds(0, TAIL)], gsems[0])
    pltpu.make_async_copy(h2_hbm.at[srcidx_v.at[pl.ds(NCF * CHUNK, TAIL)]],
                          bufs[0].at[pl.ds(0, TAIL)], gsems[0]).wait()
    pltpu.sync_copy(bufs[0].at[pl.ds(0, TAIL)],
                    acc_sh.at[dstidx_v.at[pl.ds(NCF * CHUNK, TAIL)]],
                    add=True)
    plsc.subcore_barrier()
    pltpu.sync_copy(acc_sh.at[pl.ds(s * ROWS_A, ROWS_A)],
                    out_hbm.at[c].at[pl.ds(s * ROWS_A, ROWS_A)])

    @pl.when(s == 0)
    def _():
        pltpu.sync_copy(acc_sh.at[pl.ds(NS * ROWS_A, REM)],
                        out_hbm.at[c].at[pl.ds(NS * ROWS_A, REM)])


def _t0_body(x_ref, w_ref, degp_ref, h2_ref):
    deg = degp_ref[0][:, :1] + degp_ref[1][:, :1] + 1.0
    dinv = lax.rsqrt(jnp.maximum(deg, 1.0))
    h = jnp.dot(x_ref[...], w_ref[...], preferred_element_type=jnp.float32)
    h2_ref[...] = h * dinv


def _mid_body(accp_ref, h2p_ref, degp_ref, b_ref, g_ref, be_ref,
              w_ref, hact_ref, h2_ref):
    deg = degp_ref[0][:, :1] + degp_ref[1][:, :1] + 1.0
    dinv = lax.rsqrt(jnp.maximum(deg, 1.0))
    t = (accp_ref[0] + accp_ref[1] + h2p_ref[...]) * dinv + b_ref[...]
    mean = jnp.mean(t, axis=0, keepdims=True)
    var = jnp.mean((t - mean) ** 2, axis=0, keepdims=True)
    hact = jnp.maximum((t - mean) * lax.rsqrt(var + EPS) * g_ref[...]
                       + be_ref[...], 0.0)
    hact_ref[...] = hact
    h2_ref[...] = jnp.dot(hact, w_ref[...],
                          preferred_element_type=jnp.float32) * dinv


def _fin_body(accp_ref, h2p_ref, degp_ref, b_ref, g_ref, be_ref,
              hact1_ref, hact2_ref, wo_ref, bo_ref, out_ref):
    deg = degp_ref[0][:, :1] + degp_ref[1][:, :1] + 1.0
    dinv = lax.rsqrt(jnp.maximum(deg, 1.0))
    t = (accp_ref[0] + accp_ref[1] + h2p_ref[...]) * dinv + b_ref[...]
    mean = jnp.mean(t, axis=0, keepdims=True)
    var = jnp.mean((t - mean) ** 2, axis=0, keepdims=True)
    hact3 = jnp.maximum((t - mean) * lax.rsqrt(var + EPS) * g_ref[...]
                        + be_ref[...], 0.0)
    hj = jnp.maximum(jnp.maximum(hact1_ref[...], hact2_ref[...]), hact3)
    o = jnp.dot(hj, wo_ref[...], preferred_element_type=jnp.float32) + bo_ref[...]
    m = jnp.max(o, axis=1, keepdims=True)
    sh = o - m
    lse = jnp.log(jnp.sum(jnp.exp(sh), axis=1, keepdims=True))
    out_ref[...] = sh - lse


_t0_call = pl.pallas_call(
    _t0_body,
    out_shape=jax.ShapeDtypeStruct((N, D), jnp.float32),
)

_mid_call = pl.pallas_call(
    _mid_body,
    out_shape=[jax.ShapeDtypeStruct((N, D), jnp.float32),
               jax.ShapeDtypeStruct((N, D), jnp.float32)],
)

_fin_call = pl.pallas_call(
    _fin_body,
    out_shape=jax.ShapeDtypeStruct((N, D), jnp.float32),
)


def kernel(x, adj_m, W0, b0, g0, be0, W1, b1, g1, be1, W2, b2, g2, be2, Wo, bo):
    degp = _make_deg_kernel()(adj_m)
    h2 = _t0_call(x, W0, degp)

    hacts = []
    for (b, g, be, Wn) in ((b0, g0, be0, W1), (b1, g1, be1, W2)):
        accp = _make_edge_kernel()(h2, adj_m)
        hact, h2 = _mid_call(accp, h2, degp, b, g, be, Wn)
        hacts.append(hact)

    accp = _make_edge_kernel()(h2, adj_m)
    return _fin_call(accp, h2, degp, b2, g2, be2, hacts[0], hacts[1], Wo, bo)
